# baseline (speedup 1.0000x reference)
import functools

import jax
import jax.numpy as jnp
import numpy as np
from jax import lax
from jax.experimental import pallas as pl
from jax.experimental.pallas import tpu as pltpu
from jax.experimental.pallas import tpu_sc as plsc

C = 2048
E = 131072
D = 32
H = 32
K = 2
MSG_H = 64
MSG_O = 32
TAU = 0.1

_NC = 2
_NS = 16
_NW = _NC * _NS


def _make_gather(num_tables, chunk=512):
    per_w = E // _NW
    n_chunks = per_w // chunk
    n_groups = chunk // 16
    mesh = plsc.VectorSubcoreMesh(core_axis_name="c", subcore_axis_name="s")
    out_type = [jax.ShapeDtypeStruct((E, D), jnp.float32)] * (2 * num_tables)
    scratch = [
        pltpu.VMEM((C, D), jnp.float32),
        pltpu.VMEM((chunk,), jnp.int32),
        pltpu.VMEM((chunk,), jnp.int32),
        pltpu.VMEM((chunk, D), jnp.float32),
    ]

    @functools.partial(pl.kernel, out_type=out_type, mesh=mesh,
                       scratch_types=scratch,
                       compiler_params=pltpu.CompilerParams(
                           use_tc_tiling_on_sc=False,
                           needs_layout_passes=False))
    def gather_kernel(*refs):
        tables = refs[:num_tables]
        send, rec = refs[num_tables], refs[num_tables + 1]
        outs = refs[num_tables + 2:3 * num_tables + 2]
        table_v, sidx, ridx, obuf = refs[3 * num_tables + 2:3 * num_tables + 6]
        wid = lax.axis_index("s") * _NC + lax.axis_index("c")
        base = wid * per_w
        lane = lax.iota(jnp.int32, 16)

        for ti in range(num_tables):
            pltpu.sync_copy(tables[ti], table_v)

            def chunk_step(t, carry, ti=ti):
                off = base + t * chunk
                pltpu.sync_copy(send.at[pl.ds(off, chunk)], sidx)
                pltpu.sync_copy(rec.at[pl.ds(off, chunk)], ridx)
                for which, idx_ref in ((0, sidx), (1, ridx)):
                    @plsc.parallel_loop(0, n_groups, unroll=4)
                    def group_step(g, idx_ref=idx_ref):
                        rows = idx_ref[pl.ds(g * 16, 16)]
                        orow = lane + g * 16
                        for j in range(D):
                            jcol = jnp.full((16,), j, jnp.int32)
                            vals = plsc.load_gather(table_v, [rows, jcol])
                            plsc.store_scatter(obuf, [orow, jcol], vals)
                    pltpu.sync_copy(obuf, outs[2 * ti + which].at[pl.ds(off, chunk)])
                return carry

            lax.fori_loop(0, n_chunks, chunk_step, 0)

    return gather_kernel


_gather2 = _make_gather(2)
_gather1 = _make_gather(1)


def _make_graphs_scatter(ch=2048):
    n_chunks = E // ch
    n_groups = ch // 16
    rounds = C // (16 * _NW)
    mesh = plsc.VectorSubcoreMesh(core_axis_name="c", subcore_axis_name="s")
    out_type = [jax.ShapeDtypeStruct((K, C, C), jnp.float32)]
    scratch = [
        pltpu.VMEM((16, C), jnp.float32),
        pltpu.VMEM((16, C), jnp.float32),
        pltpu.VMEM((ch,), jnp.int32), pltpu.VMEM((ch,), jnp.int32),
        pltpu.VMEM((ch,), jnp.int32), pltpu.VMEM((ch,), jnp.int32),
        pltpu.VMEM((ch,), jnp.float32), pltpu.VMEM((ch,), jnp.float32),
        pltpu.VMEM((ch,), jnp.float32), pltpu.VMEM((ch,), jnp.float32),
        pltpu.SemaphoreType.DMA, pltpu.SemaphoreType.DMA,
    ]

    @functools.partial(pl.kernel, out_type=out_type, mesh=mesh,
                       scratch_types=scratch,
                       compiler_params=pltpu.CompilerParams(
                           use_tc_tiling_on_sc=False,
                           needs_layout_passes=False))
    def scatter_kernel(send, rec, e0, e1, graphs,
                       reg0, reg1, sa, sb, ra, rb_,
                       va0, vb0, va1, vb1, sem_a, sem_b):
        svs = (sa, sb)
        rvs = (ra, rb_)
        v0s = (va0, vb0)
        v1s = (va1, vb1)
        sems = (sem_a, sem_b)
        wid = lax.axis_index("s") * _NC + lax.axis_index("c")
        lane = lax.iota(jnp.int32, 16)
        lanef = lane.astype(jnp.float32) - 16.0
        zero16 = jnp.zeros((16,), jnp.float32)

        def fire(c_idx, b):
            off = c_idx * ch
            pltpu.async_copy(send.at[pl.ds(off, ch)], svs[b], sems[b])
            pltpu.async_copy(rec.at[pl.ds(off, ch)], rvs[b], sems[b])
            pltpu.async_copy(e0.at[pl.ds(off, ch)], v0s[b], sems[b])
            pltpu.async_copy(e1.at[pl.ds(off, ch)], v1s[b], sems[b])

        def drain(b):
            pltpu.make_async_copy(send.at[pl.ds(0, ch)], svs[b], sems[b]).wait()
            pltpu.make_async_copy(rec.at[pl.ds(0, ch)], rvs[b], sems[b]).wait()
            pltpu.make_async_copy(e0.at[pl.ds(0, ch)], v0s[b], sems[b]).wait()
            pltpu.make_async_copy(e1.at[pl.ds(0, ch)], v1s[b], sems[b]).wait()

        for r in range(rounds):
            lo = (r * _NW + wid) * 16

            @plsc.parallel_loop(0, 16 * C // 16, unroll=8)
            def zstep(j):
                row = j >> 7
                col = (j & 127) * 16
                reg0[row, pl.ds(col, 16)] = zero16
                reg1[row, pl.ds(col, 16)] = zero16

            def process(c_rel, b, lo=lo):
                def gstep(g, carry):
                    s = svs[b][pl.ds(g * 16, 16)]
                    valid = (s >= lo) & (s < lo + 16)

                    def dowork():
                        rr = rvs[b][pl.ds(g * 16, 16)]
                        val0 = v0s[b][pl.ds(g * 16, 16)]
                        val1 = v1s[b][pl.ds(g * 16, 16)]
                        rowv = jnp.clip(s - lo, 0, 15)
                        plsc.store_scatter(reg0, [rowv, rr], lanef, mask=valid)
                        rbf = plsc.load_gather(reg0, [rowv, rr], mask=valid)
                        win = valid & (rbf == lanef)
                        plsc.store_scatter(reg0, [rowv, rr], val0, mask=win)
                        plsc.store_scatter(reg1, [rowv, rr], val1, mask=win)
                        anydup = jnp.any(valid & jnp.logical_not(win))

                        def slow():
                            def sstep(j, carry2):
                                mj = valid & (lane == j)
                                plsc.store_scatter(reg0, [rowv, rr], val0, mask=mj)
                                plsc.store_scatter(reg1, [rowv, rr], val1, mask=mj)
                                return carry2
                            lax.fori_loop(0, 16, sstep, 0)

                        lax.cond(anydup, slow, lambda: None)

                    lax.cond(jnp.any(valid), dowork, lambda: None)
                    return carry

                lax.fori_loop(0, n_groups, gstep, 0)

            fire(0, 0)

            def pairstep(t, carry):
                c0 = 2 * t
                fire(c0 + 1, 1)
                drain(0)
                process(c0, 0)
                fire(jnp.minimum(c0 + 2, n_chunks - 1), 0)
                drain(1)
                process(c0 + 1, 1)
                return carry

            lax.fori_loop(0, n_chunks // 2, pairstep, 0)
            drain(0)
            pltpu.sync_copy(reg0, graphs.at[0, pl.ds(lo, 16)])
            pltpu.sync_copy(reg1, graphs.at[1, pl.ds(lo, 16)])

    return scatter_kernel


_graphs_scatter = _make_graphs_scatter()


def _mlp(x, p, name):
    x = jax.nn.relu(x @ p[name + '_w1'] + p[name + '_b1'])
    x = jax.nn.relu(x @ p[name + '_w2'] + p[name + '_b2'])
    mean = jnp.mean(x, axis=0, keepdims=True)
    var = jnp.var(x, axis=0, keepdims=True)
    x = (x - mean) / jnp.sqrt(var + 1e-5)
    return x * p[name + '_g'] + p[name + '_be']


def _head_kernel(agg_ref, w1_ref, b1_ref, w2_ref, b2_ref, out_ref):
    pred = jnp.maximum(agg_ref[...] @ w1_ref[...] + b1_ref[...], 0.0)
    out_ref[...] = pred @ w2_ref[...] + b2_ref[...]


def kernel(data, params, send_idx, rec_idx):
    p = params
    x1 = _mlp(data, p, 'enc1')
    xs1, xr1, ds0, dr0 = _gather2(x1, data, send_idx, rec_idx)
    x = jnp.concatenate([xs1, xr1], axis=-1)
    x = _mlp(x, p, 'enc2')
    x_skip = x
    x = jax.ops.segment_sum(x, rec_idx, num_segments=C) / C
    x3 = _mlp(x, p, 'enc3')
    xs3, xr3 = _gather1(x3, send_idx, rec_idx)
    x = jnp.concatenate([xs3, xr3, x_skip], axis=-1)
    x = _mlp(x, p, 'enc4')
    logits = x @ p['fc_out_w'] + p['fc_out_b']
    u = jax.random.uniform(jax.random.key(42), logits.shape, minval=1e-6, maxval=1.0 - 1e-6)
    g = -jnp.log(-jnp.log(u))
    edges = jax.nn.softmax((logits + g) / TAU, axis=-1)
    prob = jax.nn.softmax(logits, axis=-1)

    pre_msg = jnp.concatenate([ds0, dr0], axis=-1)
    all_msgs = jnp.zeros((E, MSG_O), jnp.float32)
    for i in range(K):
        m = jax.nn.relu(pre_msg @ p['msg1_%d_w' % i] + p['msg1_%d_b' % i])
        m = jax.nn.relu(m @ p['msg2_%d_w' % i] + p['msg2_%d_b' % i])
        all_msgs = all_msgs + m * edges[:, i:i + 1]
    agg = jax.ops.segment_sum(all_msgs, rec_idx, num_segments=C) / C

    output = pl.pallas_call(
        _head_kernel,
        out_shape=jax.ShapeDtypeStruct((C, D), jnp.float32),
    )(agg, p['out1_w'], p['out1_b'], p['out2_w'], p['out2_b'])

    e0 = edges[:, 0] + 0.0
    e1 = edges[:, 1] + 0.0
    graphs = _graphs_scatter(send_idx, rec_idx, e0, e1)[0]
    return graphs, output, prob


# speedup vs baseline: 1.2971x; 1.2971x over previous
import functools

import jax
import jax.numpy as jnp
import numpy as np
from jax import lax
from jax.experimental import pallas as pl
from jax.experimental.pallas import tpu as pltpu
from jax.experimental.pallas import tpu_sc as plsc

C = 2048
E = 131072
D = 32
H = 32
K = 2
MSG_H = 64
MSG_O = 32
TAU = 0.1

_NC = 2
_NS = 16
_NW = _NC * _NS


def _make_gather(num_tables, chunk=512):
    per_w = E // _NW
    n_chunks = per_w // chunk
    n_groups = chunk // 16
    mesh = plsc.VectorSubcoreMesh(core_axis_name="c", subcore_axis_name="s")
    out_type = [jax.ShapeDtypeStruct((E, D), jnp.float32)] * (2 * num_tables)
    scratch = [
        pltpu.VMEM((C, D), jnp.float32),
        pltpu.VMEM((chunk,), jnp.int32),
        pltpu.VMEM((chunk,), jnp.int32),
        pltpu.VMEM((chunk, D), jnp.float32),
    ]

    @functools.partial(pl.kernel, out_type=out_type, mesh=mesh,
                       scratch_types=scratch,
                       compiler_params=pltpu.CompilerParams(
                           use_tc_tiling_on_sc=False,
                           needs_layout_passes=False))
    def gather_kernel(*refs):
        tables = refs[:num_tables]
        send, rec = refs[num_tables], refs[num_tables + 1]
        outs = refs[num_tables + 2:3 * num_tables + 2]
        table_v, sidx, ridx, obuf = refs[3 * num_tables + 2:3 * num_tables + 6]
        wid = lax.axis_index("s") * _NC + lax.axis_index("c")
        base = wid * per_w
        lane = lax.iota(jnp.int32, 16)

        for ti in range(num_tables):
            pltpu.sync_copy(tables[ti], table_v)

            def chunk_step(t, carry, ti=ti):
                off = base + t * chunk
                pltpu.sync_copy(send.at[pl.ds(off, chunk)], sidx)
                pltpu.sync_copy(rec.at[pl.ds(off, chunk)], ridx)
                for which, idx_ref in ((0, sidx), (1, ridx)):
                    @plsc.parallel_loop(0, n_groups, unroll=4)
                    def group_step(g, idx_ref=idx_ref):
                        rows = idx_ref[pl.ds(g * 16, 16)]
                        orow = lane + g * 16
                        for j in range(D):
                            jcol = jnp.full((16,), j, jnp.int32)
                            vals = plsc.load_gather(table_v, [rows, jcol])
                            plsc.store_scatter(obuf, [orow, jcol], vals)
                    pltpu.sync_copy(obuf, outs[2 * ti + which].at[pl.ds(off, chunk)])
                return carry

            lax.fori_loop(0, n_chunks, chunk_step, 0)

    return gather_kernel


_gather2 = _make_gather(2)
_gather1 = _make_gather(1)


def _make_graphs_scatter(ch=2048):
    n_chunks = E // ch
    n_groups = ch // 16
    rounds = C // (16 * _NW)
    mesh = plsc.VectorSubcoreMesh(core_axis_name="c", subcore_axis_name="s")
    out_type = [jax.ShapeDtypeStruct((K, C, C), jnp.float32)]
    scratch = [
        pltpu.VMEM((16, C), jnp.float32),
        pltpu.VMEM((16, C), jnp.float32),
        pltpu.VMEM((16, C), jnp.int32),
        pltpu.VMEM((ch,), jnp.int32), pltpu.VMEM((ch,), jnp.int32),
        pltpu.VMEM((ch,), jnp.int32), pltpu.VMEM((ch,), jnp.int32),
        pltpu.VMEM((ch,), jnp.float32), pltpu.VMEM((ch,), jnp.float32),
        pltpu.VMEM((ch,), jnp.float32), pltpu.VMEM((ch,), jnp.float32),
        pltpu.SemaphoreType.DMA, pltpu.SemaphoreType.DMA,
    ]

    @functools.partial(pl.kernel, out_type=out_type, mesh=mesh,
                       scratch_types=scratch,
                       compiler_params=pltpu.CompilerParams(
                           use_tc_tiling_on_sc=False,
                           needs_layout_passes=False))
    def scatter_kernel(send, rec, e0, e1, graphs,
                       reg0, reg1, tmp, sa, sb, ra, rb_,
                       va0, vb0, va1, vb1, sem_a, sem_b):
        svs = (sa, sb)
        rvs = (ra, rb_)
        v0s = (va0, vb0)
        v1s = (va1, vb1)
        sems = (sem_a, sem_b)
        wid = lax.axis_index("s") * _NC + lax.axis_index("c")
        lane = lax.iota(jnp.int32, 16)
        zero16 = jnp.zeros((16,), jnp.float32)

        def fire(c_idx, b):
            off = c_idx * ch
            pltpu.async_copy(send.at[pl.ds(off, ch)], svs[b], sems[b])
            pltpu.async_copy(rec.at[pl.ds(off, ch)], rvs[b], sems[b])
            pltpu.async_copy(e0.at[pl.ds(off, ch)], v0s[b], sems[b])
            pltpu.async_copy(e1.at[pl.ds(off, ch)], v1s[b], sems[b])

        def drain(b):
            pltpu.make_async_copy(send.at[pl.ds(0, ch)], svs[b], sems[b]).wait()
            pltpu.make_async_copy(rec.at[pl.ds(0, ch)], rvs[b], sems[b]).wait()
            pltpu.make_async_copy(e0.at[pl.ds(0, ch)], v0s[b], sems[b]).wait()
            pltpu.make_async_copy(e1.at[pl.ds(0, ch)], v1s[b], sems[b]).wait()

        for r in range(rounds):
            lo = (r * _NW + wid) * 16

            @plsc.parallel_loop(0, 16 * C // 16, unroll=8)
            def zstep(j):
                row = j >> 7
                col = (j & 127) * 16
                reg0[row, pl.ds(col, 16)] = zero16
                reg1[row, pl.ds(col, 16)] = zero16

            def process(c_rel, b, lo=lo):
                def gstep(g, carry):
                    s = svs[b][pl.ds(g * 16, 16)]
                    valid = (s >= lo) & (s < lo + 16)

                    def dowork():
                        rr = rvs[b][pl.ds(g * 16, 16)]
                        val0 = v0s[b][pl.ds(g * 16, 16)]
                        val1 = v1s[b][pl.ds(g * 16, 16)]
                        rowv = jnp.clip(s - lo, 0, 15)
                        plsc.store_scatter(tmp, [rowv, rr], lane, mask=valid)
                        rb = plsc.load_gather(tmp, [rowv, rr], mask=valid)
                        anydup = jnp.any(valid & (rb != lane))

                        def fast():
                            ok = valid & (rb == lane)
                            plsc.store_scatter(reg0, [rowv, rr], val0, mask=ok)
                            plsc.store_scatter(reg1, [rowv, rr], val1, mask=ok)

                        def slow():
                            def sstep(j, carry2):
                                mj = valid & (lane == j)
                                plsc.store_scatter(reg0, [rowv, rr], val0, mask=mj)
                                plsc.store_scatter(reg1, [rowv, rr], val1, mask=mj)
                                return carry2
                            lax.fori_loop(0, 16, sstep, 0)

                        lax.cond(anydup, slow, fast)

                    lax.cond(jnp.any(valid), dowork, lambda: None)
                    return carry

                lax.fori_loop(0, n_groups, gstep, 0)

            fire(0, 0)

            def pairstep(t, carry):
                c0 = 2 * t
                fire(c0 + 1, 1)
                drain(0)
                process(c0, 0)
                fire(jnp.minimum(c0 + 2, n_chunks - 1), 0)
                drain(1)
                process(c0 + 1, 1)
                return carry

            lax.fori_loop(0, n_chunks // 2, pairstep, 0)
            drain(0)
            pltpu.sync_copy(reg0, graphs.at[0, pl.ds(lo, 16)])
            pltpu.sync_copy(reg1, graphs.at[1, pl.ds(lo, 16)])

    return scatter_kernel


_graphs_scatter = _make_graphs_scatter()


def _mlp(x, p, name):
    x = jax.nn.relu(x @ p[name + '_w1'] + p[name + '_b1'])
    x = jax.nn.relu(x @ p[name + '_w2'] + p[name + '_b2'])
    mean = jnp.mean(x, axis=0, keepdims=True)
    var = jnp.var(x, axis=0, keepdims=True)
    x = (x - mean) / jnp.sqrt(var + 1e-5)
    return x * p[name + '_g'] + p[name + '_be']


def _head_kernel(agg_ref, w1_ref, b1_ref, w2_ref, b2_ref, out_ref):
    pred = jnp.maximum(agg_ref[...] @ w1_ref[...] + b1_ref[...], 0.0)
    out_ref[...] = pred @ w2_ref[...] + b2_ref[...]


def kernel(data, params, send_idx, rec_idx):
    p = params
    x1 = _mlp(data, p, 'enc1')
    xs1, xr1, ds0, dr0 = _gather2(x1, data, send_idx, rec_idx)
    x = jnp.concatenate([xs1, xr1], axis=-1)
    x = _mlp(x, p, 'enc2')
    x_skip = x
    x = jax.ops.segment_sum(x, rec_idx, num_segments=C) / C
    x3 = _mlp(x, p, 'enc3')
    xs3, xr3 = _gather1(x3, send_idx, rec_idx)
    x = jnp.concatenate([xs3, xr3, x_skip], axis=-1)
    x = _mlp(x, p, 'enc4')
    logits = x @ p['fc_out_w'] + p['fc_out_b']
    u = jax.random.uniform(jax.random.key(42), logits.shape, minval=1e-6, maxval=1.0 - 1e-6)
    g = -jnp.log(-jnp.log(u))
    edges = jax.nn.softmax((logits + g) / TAU, axis=-1)
    prob = jax.nn.softmax(logits, axis=-1)

    pre_msg = jnp.concatenate([ds0, dr0], axis=-1)
    all_msgs = jnp.zeros((E, MSG_O), jnp.float32)
    for i in range(K):
        m = jax.nn.relu(pre_msg @ p['msg1_%d_w' % i] + p['msg1_%d_b' % i])
        m = jax.nn.relu(m @ p['msg2_%d_w' % i] + p['msg2_%d_b' % i])
        all_msgs = all_msgs + m * edges[:, i:i + 1]
    agg = jax.ops.segment_sum(all_msgs, rec_idx, num_segments=C) / C

    output = pl.pallas_call(
        _head_kernel,
        out_shape=jax.ShapeDtypeStruct((C, D), jnp.float32),
    )(agg, p['out1_w'], p['out1_b'], p['out2_w'], p['out2_b'])

    e0 = edges[:, 0] + 0.0
    e1 = edges[:, 1] + 0.0
    graphs = _graphs_scatter(send_idx, rec_idx, e0, e1)[0]
    return graphs, output, prob
